# Initial kernel scaffold; baseline (speedup 1.0000x reference)
import jax
import jax.numpy as jnp
from jax.experimental import pallas as pl


def kernel(x, edge_index, W1, b1, W2, b2):
    raise NotImplementedError("write your pallas kernel here")


# speedup vs baseline: 37.0405x; 37.0405x over previous
import functools

import jax
import jax.numpy as jnp
from jax import lax
from jax.experimental import pallas as pl
from jax.experimental.pallas import tpu as pltpu
from jax.experimental.pallas import tpu_sc as plsc

N = 10000
E = 320000
IN_CH = 128
HID_CH = 32
OUT_CH = 16

NC, NS = 2, 16
NW = NC * NS
EPW = E // NW
CHUNK = 80
NCHUNK = EPW // CHUNK
N_PAD = 10240
DEG_PAD = N_PAD
DPT = DEG_PAD // NS
RPT = N_PAD // NS

_mesh = plsc.VectorSubcoreMesh(core_axis_name="c", subcore_axis_name="s",
                               num_cores=NC, num_subcores=NS)
_sc_params = pltpu.CompilerParams(use_tc_tiling_on_sc=False)


@functools.partial(
    pl.kernel,
    out_type=jax.ShapeDtypeStruct((NC, DEG_PAD), jnp.float32),
    mesh=_mesh,
    compiler_params=_sc_params,
    scratch_types=[
        pltpu.VMEM((NCHUNK, CHUNK), jnp.int32),
        pltpu.VMEM((CHUNK,), jnp.float32),
        pltpu.VMEM((DPT,), jnp.float32),
        pltpu.VMEM_SHARED((DEG_PAD,), jnp.float32),
    ],
)
def _deg_kernel(col_hbm, deg_hbm, col_v, ones_v, zb_v, deg_sh):
    c = lax.axis_index("c")
    s = lax.axis_index("s")
    wid = c * NS + s
    pltpu.sync_copy(col_hbm.at[wid], col_v)
    for i in range(CHUNK // 16):
        ones_v[pl.ds(16 * i, 16)] = jnp.ones((16,), jnp.float32)
    for i in range(DPT // 16):
        zb_v[pl.ds(16 * i, 16)] = jnp.zeros((16,), jnp.float32)
    pltpu.sync_copy(zb_v, deg_sh.at[pl.ds(s * DPT, DPT)])
    plsc.subcore_barrier()

    @pl.loop(0, NCHUNK)
    def _(j):
        pltpu.sync_copy(ones_v, deg_sh.at[col_v.at[j]], add=True)

    plsc.subcore_barrier()
    pltpu.sync_copy(deg_sh.at[pl.ds(s * DPT, DPT)],
                    deg_hbm.at[c, pl.ds(s * DPT, DPT)])


def _make_agg_kernel(d):
    @functools.partial(
        pl.kernel,
        out_type=jax.ShapeDtypeStruct((NC, N_PAD, d), jnp.float32),
        mesh=_mesh,
        compiler_params=_sc_params,
        scratch_types=[
            pltpu.VMEM((NCHUNK, CHUNK), jnp.int32),
            pltpu.VMEM((NCHUNK, CHUNK), jnp.int32),
            pltpu.VMEM((CHUNK, d), jnp.float32),
            pltpu.VMEM((CHUNK, d), jnp.float32),
            pltpu.VMEM_SHARED((N_PAD, d), jnp.float32),
            pltpu.SemaphoreType.DMA,
            pltpu.SemaphoreType.DMA,
        ],
    )
    def k(z_hbm, zeros_hbm, row_hbm, col_hbm, agg_hbm,
          row_v, col_v, msg0, msg1, agg_sh, sem0, sem1):
        c = lax.axis_index("c")
        s = lax.axis_index("s")
        wid = c * NS + s
        cp_r = pltpu.async_copy(row_hbm.at[wid], row_v, sem0)
        cp_c = pltpu.async_copy(col_hbm.at[wid], col_v, sem1)
        pltpu.sync_copy(zeros_hbm.at[pl.ds(s * RPT, RPT)],
                        agg_sh.at[pl.ds(s * RPT, RPT)])
        cp_r.wait()
        cp_c.wait()
        plsc.subcore_barrier()

        @pl.loop(0, NCHUNK // 2)
        def _(i):
            j = 2 * i
            g0 = pltpu.async_copy(z_hbm.at[row_v.at[j]], msg0, sem0)
            g1 = pltpu.async_copy(z_hbm.at[row_v.at[j + 1]], msg1, sem1)
            g0.wait()
            pltpu.sync_copy(msg0, agg_sh.at[col_v.at[j]], add=True)
            g1.wait()
            pltpu.sync_copy(msg1, agg_sh.at[col_v.at[j + 1]], add=True)

        if NCHUNK % 2:
            j = NCHUNK - 1
            pltpu.async_copy(z_hbm.at[row_v.at[j]], msg0, sem0).wait()
            pltpu.sync_copy(msg0, agg_sh.at[col_v.at[j]], add=True)

        plsc.subcore_barrier()
        pltpu.sync_copy(agg_sh.at[pl.ds(s * RPT, RPT)],
                        agg_hbm.at[c, pl.ds(s * RPT, RPT)])

    return k


_agg_hid = _make_agg_kernel(HID_CH)
_agg_out = _make_agg_kernel(OUT_CH)


def _scale_in_kernel(x, W1, degt):
    def body(x_ref, w_ref, deg_ref, z_ref, dis_ref):
        deg = 1.0 + deg_ref[:, 0:1] + deg_ref[:, 1:2]
        dis = lax.rsqrt(deg)
        xw = jnp.dot(x_ref[...], w_ref[...], preferred_element_type=jnp.float32)
        z_ref[...] = dis * xw
        dis_ref[...] = dis

    return pl.pallas_call(
        body,
        out_shape=(jax.ShapeDtypeStruct((N, HID_CH), jnp.float32),
                   jax.ShapeDtypeStruct((N, 1), jnp.float32)),
    )(x, W1, degt)


def _mid_kernel(agg1, z1, dis, W2, b1):
    def body(p_ref, z_ref, dis_ref, w_ref, b_ref, z2_ref):
        agg = p_ref[0] + p_ref[1] + z_ref[...]
        h = jnp.maximum(dis_ref[...] * agg + b_ref[...], 0.0)
        xw2 = jnp.dot(h, w_ref[...], preferred_element_type=jnp.float32)
        z2_ref[...] = dis_ref[...] * xw2

    return pl.pallas_call(
        body,
        out_shape=jax.ShapeDtypeStruct((N, OUT_CH), jnp.float32),
    )(agg1, z1, dis, W2, b1)


def _final_kernel(agg2, z2, dis, b2):
    def body(q_ref, z2_ref, dis_ref, b_ref, out_ref):
        out_ref[...] = dis_ref[...] * (q_ref[0] + q_ref[1] + z2_ref[...]) + b_ref[...]

    return pl.pallas_call(
        body,
        out_shape=jax.ShapeDtypeStruct((N, OUT_CH), jnp.float32),
    )(agg2, z2, dis, b2)


def kernel(x, edge_index, W1, b1, W2, b2):
    row = edge_index[0].astype(jnp.int32).reshape(NW, NCHUNK, CHUNK)
    col = edge_index[1].astype(jnp.int32).reshape(NW, NCHUNK, CHUNK)

    degp = _deg_kernel(col)
    degt = degp[:, :N].T
    z1, dis = _scale_in_kernel(x, W1, degt)

    zeros_hid = jnp.zeros((N_PAD, HID_CH), jnp.float32)
    agg1 = _agg_hid(z1, zeros_hid, row, col)[:, :N]
    z2 = _mid_kernel(agg1, z1, dis, W2, b1.reshape(1, HID_CH))

    zeros_out = jnp.zeros((N_PAD, OUT_CH), jnp.float32)
    agg2 = _agg_out(z2, zeros_out, row, col)[:, :N]
    return _final_kernel(agg2, z2, dis, b2.reshape(1, OUT_CH))
